# baseline (speedup 1.0000x reference)
import functools

import jax
import jax.numpy as jnp
from jax import lax
from jax.experimental import pallas as pl
from jax.experimental.pallas import tpu as pltpu
from jax.experimental.pallas import tpu_sc as plsc

B = 16384
L = 50
EMBED = 64
TOTAL = B * L

_info = plsc.get_sparse_core_info()
NC = _info.num_cores
NS = _info.num_subcores
NW = NC * NS

NB = 2
GROUP = NB * 128
N_UNITS = L * (B // 128)
N_GROUPS = N_UNITS // NB
G_PER_W = N_GROUPS // NW
ACC = NB * 1024
NR = 3

_mesh = plsc.VectorSubcoreMesh(core_axis_name="c", subcore_axis_name="s")


@functools.partial(
    pl.kernel,
    out_type=jax.ShapeDtypeStruct((L * EMBED * B,), jnp.float32),
    mesh=_mesh,
    scratch_types=[
        [pltpu.VMEM((GROUP,), jnp.int32) for _ in range(NR)],
        [pltpu.VMEM((GROUP, EMBED), jnp.float32) for _ in range(NR)],
        [pltpu.VMEM((8 * ACC,), jnp.float32) for _ in range(2)],
        [pltpu.SemaphoreType.DMA for _ in range(NR)],
        [pltpu.SemaphoreType.DMA for _ in range(NR)],
        [pltpu.SemaphoreType.DMA for _ in range(2)],
    ],
    compiler_params=pltpu.CompilerParams(
        use_tc_tiling_on_sc=False, needs_layout_passes=False
    ),
)
def _embed(idx_hbm, table_hbm, out_hbm, idx_v, rows, acc, sem_i, sem_g, sem_w):
    wid = lax.axis_index("s") * NC + lax.axis_index("c")
    g0 = wid * G_PER_W

    lane = lax.iota(jnp.int32, 16)
    f0lane = []
    fbase = []
    for k in range(4):
        f = lane + 16 * k
        f0lane.append(f)
        fbase.append((f >> 3) * ACC + (f & 7) * 128)

    def unit(g):
        u0 = (g0 + g) * NB
        return u0 // 128, u0 % 128

    def start_idx(g, b):
        gg = jnp.minimum(g, G_PER_W - 1)
        l, bg0 = unit(gg)
        pltpu.async_copy(
            idx_hbm.at[l, pl.ds(bg0 * 128, GROUP)], idx_v[b], sem_i[b]
        )

    def wait_idx(b):
        pltpu.make_async_copy(
            idx_hbm.at[0, pl.ds(0, GROUP)], idx_v[b], sem_i[b]
        ).wait()

    def start_gather(b):
        pltpu.async_copy(table_hbm.at[idx_v[b]], rows[b], sem_g[b])

    def wait_gather(b):
        pltpu.make_async_copy(
            table_hbm.at[pl.ds(0, GROUP)], rows[b], sem_g[b]
        ).wait()

    def transpose(rb, ab):
        def body(t, _):
            j0 = t * 16
            jbase = (j0 >> 7) * 1024 + (j0 & 127)
            for s in range(16):
                d = (lane + s) & 15
                jvec = j0 + d
                jd = jbase + d
                for k in range(4):
                    v = plsc.load_gather(rows[rb], [jvec, f0lane[k]])
                    plsc.store_scatter(acc[ab], [fbase[k] + jd], v)
            return ()

        lax.fori_loop(0, GROUP // 16, body, ())

    def start_writes(g, ab):
        l, bg0 = unit(g)
        for fg in range(8):
            pltpu.async_copy(
                acc[ab].at[pl.ds(fg * ACC, ACC)],
                out_hbm.at[pl.ds(((l * 8 + fg) * 128 + bg0) * 1024, ACC)],
                sem_w[ab],
            )

    def wait_writes(ab):
        for fg in range(8):
            pltpu.make_async_copy(
                acc[ab].at[pl.ds(fg * ACC, ACC)],
                out_hbm.at[pl.ds(fg * ACC, ACC)],
                sem_w[ab],
            ).wait()

    start_idx(0, 0)
    wait_idx(0)
    start_gather(0)
    start_idx(1, 1)
    start_idx(2, 2)
    wait_idx(1)
    start_gather(1)

    def body(g, _):
        m = lax.rem(g, 6)

        def step(rb, rb2, ab):
            wait_gather(rb)
            wait_idx(rb2)
            start_gather(rb2)
            start_idx(g + 3, rb)

            @pl.when(g >= 2)
            def _():
                wait_writes(ab)

            transpose(rb, ab)
            start_writes(g, ab)

        for c in range(6):
            @pl.when(m == c)
            def _(c=c):
                step(c % 3, (c + 2) % 3, c % 2)

        return ()

    lax.fori_loop(0, G_PER_W, body, ())

    wait_gather((G_PER_W + 0) % 3)
    wait_gather((G_PER_W + 1) % 3)
    wait_idx((G_PER_W - 1) % 3)
    wait_writes(0)
    wait_writes(1)


def kernel(inputs, table):
    idx_t = inputs.T.astype(jnp.int32)
    flat = _embed(idx_t, table)
    out5 = flat.reshape(L, 8, 128, 8, 128)
    return out5.transpose(2, 4, 0, 1, 3).reshape(B, L, EMBED)


# speedup vs baseline: 1.0078x; 1.0078x over previous
import functools

import jax
import jax.numpy as jnp
from jax import lax
from jax.experimental import pallas as pl
from jax.experimental.pallas import tpu as pltpu
from jax.experimental.pallas import tpu_sc as plsc

B = 16384
L = 50
EMBED = 64
TOTAL = B * L

_info = plsc.get_sparse_core_info()
NC = _info.num_cores
NS = _info.num_subcores
NW = NC * NS

NB = 2
GROUP = NB * 128
N_UNITS = L * (B // 128)
N_GROUPS = N_UNITS // NB
G_PER_W = N_GROUPS // NW
ACC = NB * 1024

_mesh = plsc.VectorSubcoreMesh(core_axis_name="c", subcore_axis_name="s")


@functools.partial(
    pl.kernel,
    out_type=jax.ShapeDtypeStruct((L * EMBED * B,), jnp.float32),
    mesh=_mesh,
    scratch_types=[
        [pltpu.VMEM((GROUP,), jnp.int32) for _ in range(2)],
        [pltpu.VMEM((GROUP, EMBED), jnp.float32) for _ in range(2)],
        [pltpu.VMEM((8 * ACC,), jnp.float32) for _ in range(2)],
        [pltpu.SemaphoreType.DMA for _ in range(2)],
        [pltpu.SemaphoreType.DMA for _ in range(2)],
        [pltpu.SemaphoreType.DMA for _ in range(2)],
    ],
    compiler_params=pltpu.CompilerParams(
        use_tc_tiling_on_sc=False, needs_layout_passes=False
    ),
)
def _embed(idx_hbm, table_hbm, out_hbm, idx_v, rows, acc, sem_i, sem_g, sem_w):
    wid = lax.axis_index("s") * NC + lax.axis_index("c")
    g0 = wid * G_PER_W

    lane = lax.iota(jnp.int32, 16)
    f0lane = []
    fbase = []
    for k in range(4):
        f = lane + 16 * k
        f0lane.append(f)
        fbase.append((f >> 3) * ACC + (f & 7) * 128)

    def unit(g):
        u0 = (g0 + g) * NB
        return u0 // 128, u0 % 128

    def start_idx(g, b):
        gg = jnp.minimum(g, G_PER_W - 1)
        l, bg0 = unit(gg)
        pltpu.async_copy(
            idx_hbm.at[l, pl.ds(bg0 * 128, GROUP)], idx_v[b], sem_i[b]
        )

    def wait_idx(b):
        pltpu.make_async_copy(
            idx_hbm.at[0, pl.ds(0, GROUP)], idx_v[b], sem_i[b]
        ).wait()

    def start_gather(b):
        pltpu.async_copy(table_hbm.at[idx_v[b]], rows[b], sem_g[b])

    def wait_gather(b):
        pltpu.make_async_copy(
            table_hbm.at[pl.ds(0, GROUP)], rows[b], sem_g[b]
        ).wait()

    def transpose(b):
        def body(t, _):
            j0 = t * 16
            jbase = (j0 >> 7) * 1024 + (j0 & 127)
            for s in range(16):
                d = (lane + s) & 15
                jvec = j0 + d
                jd = jbase + d
                for k in range(4):
                    v = plsc.load_gather(rows[b], [jvec, f0lane[k]])
                    plsc.store_scatter(acc[b], [fbase[k] + jd], v)
            return ()

        lax.fori_loop(0, GROUP // 16, body, ())

    def start_writes(g, b):
        l, bg0 = unit(g)
        for fg in range(8):
            pltpu.async_copy(
                acc[b].at[pl.ds(fg * ACC, ACC)],
                out_hbm.at[pl.ds(((l * 8 + fg) * 128 + bg0) * 1024, ACC)],
                sem_w[b],
            )

    def wait_writes(b):
        for fg in range(8):
            pltpu.make_async_copy(
                acc[b].at[pl.ds(fg * ACC, ACC)],
                out_hbm.at[pl.ds(fg * ACC, ACC)],
                sem_w[b],
            ).wait()

    start_idx(0, 0)
    wait_idx(0)
    start_gather(0)
    start_idx(1, 1)

    def body(g, _):
        b = lax.rem(g, 2)

        def even(b0, nb0):
            wait_idx(nb0)
            wait_gather(b0)
            start_gather(nb0)
            start_idx(g + 2, b0)

            @pl.when(g >= 2)
            def _():
                wait_writes(b0)

            transpose(b0)
            start_writes(g, b0)

        @pl.when(b == 0)
        def _():
            even(0, 1)

        @pl.when(b == 1)
        def _():
            even(1, 0)

        return ()

    lax.fori_loop(0, G_PER_W, body, ())

    wait_gather(0)
    wait_idx(1)
    wait_writes(0)
    wait_writes(1)


def kernel(inputs, table):
    idx_t = inputs.T.astype(jnp.int32)
    flat = _embed(idx_t, table)
    out5 = flat.reshape(L, 8, 128, 8, 128)
    return out5.transpose(2, 4, 0, 1, 3).reshape(B, L, EMBED)
